# Initial kernel scaffold; baseline (speedup 1.0000x reference)
import jax
import jax.numpy as jnp
from jax.experimental import pallas as pl


def kernel(center, context, negatives, W_in, W_out):
    raise NotImplementedError("write your pallas kernel here")


# speedup vs baseline: 4.7777x; 4.7777x over previous
import functools

import jax
import jax.numpy as jnp
from jax import lax
from jax.experimental import pallas as pl
from jax.experimental.pallas import tpu as pltpu
from jax.experimental.pallas import tpu_sc as plsc

D = 64
B = 16384
K = 20

NC, NS, L = 2, 16, 16
NW = NC * NS
PER_W = B // NW
CHUNK = 64
NSTEP = PER_W // CHUNK
NIDX_ROWS = CHUNK * K // 128


def _sc_scores_body(cen_hbm, ctx_hbm, neg_hbm, win_hbm, wout_hbm,
                    pos_out, neg_out,
                    cidx, xidx, nidx, crows, prows, nrows, psc, nsc, sem):
    wid = lax.axis_index("s") * NC + lax.axis_index("c")

    def step(c, _):
        base = wid * PER_W + c * CHUNK
        pltpu.sync_copy(cen_hbm.at[pl.ds(base, CHUNK)], cidx)
        pltpu.sync_copy(ctx_hbm.at[pl.ds(base, CHUNK)], xidx)
        pltpu.sync_copy(neg_hbm.at[pl.ds(base * K, CHUNK * K)], nidx)
        cps = [pltpu.async_copy(win_hbm.at[cidx], crows, sem),
               pltpu.async_copy(wout_hbm.at[xidx], prows, sem)]
        for j in range(NIDX_ROWS):
            cps.append(pltpu.async_copy(wout_hbm.at[nidx.at[pl.ds(j * 128, 128)]],
                                        nrows.at[pl.ds(j * 128, 128)], sem))
        for cp in cps:
            cp.wait()

        lane15 = lax.iota(jnp.int32, L) == (L - 1)

        def brow(b, _):
            cvs = [crows[b, pl.ds(j * L, L)] for j in range(D // L)]
            pvs = [prows[b, pl.ds(j * L, L)] for j in range(D // L)]
            s = plsc.cumsum(sum(c * p for c, p in zip(cvs, pvs)))
            plsc.store_scatter(psc, [jnp.full((L,), b, jnp.int32)], s,
                               mask=lane15)
            for k in range(K):
                nvs = [nrows[b * K + k, pl.ds(j * L, L)] for j in range(D // L)]
                t = plsc.cumsum(sum(c * n for c, n in zip(cvs, nvs)))
                plsc.store_scatter(nsc, [jnp.full((L,), b * K + k, jnp.int32)],
                                   t, mask=lane15)
            return 0

        lax.fori_loop(0, CHUNK, brow, 0)

        pltpu.sync_copy(psc, pos_out.at[pl.ds(base, CHUNK)])
        pltpu.sync_copy(nsc, neg_out.at[pl.ds(base * K, CHUNK * K)])
        return 0

    lax.fori_loop(0, NSTEP, step, 0)


@functools.partial(jax.jit, static_argnames=())
def _sc_scores(cen, ctx, neg2d, w_in, w_out):
    f = pl.kernel(
        _sc_scores_body,
        out_type=(jax.ShapeDtypeStruct((B,), jnp.float32),
                  jax.ShapeDtypeStruct((B * K,), jnp.float32)),
        mesh=plsc.VectorSubcoreMesh(core_axis_name="c", subcore_axis_name="s"),
        compiler_params=pltpu.CompilerParams(needs_layout_passes=False,
                                             use_tc_tiling_on_sc=False),
        scratch_types=[
            pltpu.VMEM((CHUNK,), jnp.int32),
            pltpu.VMEM((CHUNK,), jnp.int32),
            pltpu.VMEM((CHUNK * K,), jnp.int32),
            pltpu.VMEM((CHUNK, D), jnp.float32),
            pltpu.VMEM((CHUNK, D), jnp.float32),
            pltpu.VMEM((CHUNK * K, D), jnp.float32),
            pltpu.VMEM((CHUNK,), jnp.float32),
            pltpu.VMEM((CHUNK * K,), jnp.float32),
            pltpu.SemaphoreType.DMA,
        ],
    )
    return f(cen, ctx, neg2d, w_in, w_out)


def _tc_loss_body(pos_ref, neg_ref, out_ref):
    pls = jax.nn.log_sigmoid(pos_ref[...])
    nls = jax.nn.log_sigmoid(-neg_ref[...])
    out_ref[0, 0] = -(jnp.sum(pls) + jnp.sum(nls)) / B


def _tc_loss(pos2d, neg2d):
    return pl.pallas_call(
        _tc_loss_body,
        out_shape=jax.ShapeDtypeStruct((1, 1), jnp.float32),
        out_specs=pl.BlockSpec(memory_space=pltpu.SMEM),
    )(pos2d, neg2d)


def kernel(center, context, negatives, W_in, W_out):
    cen = center.astype(jnp.int32)
    ctx = context.astype(jnp.int32)
    neg = negatives.astype(jnp.int32).reshape(B * K)
    pos_s, neg_s = _sc_scores(cen, ctx, neg, W_in, W_out)
    loss = _tc_loss(pos_s.reshape(B // 128, 128),
                    neg_s.reshape(B * K // 128, 128))
    return loss[0, 0]
